# Initial kernel scaffold; baseline (speedup 1.0000x reference)
import jax
import jax.numpy as jnp
from jax.experimental import pallas as pl


def kernel(metadata_prot, metadata_mol, x_prot, edge_index_prot, batch_vector_prot, x_mol, edge_index_mol, batch_vector_mol, fc1_W, fc1_b, fc2_W, fc2_b, fcm1_W, fcm1_b, fcm2_W, fcm2_b, gp1_W, gp1_b, gp2_W, gp2_b, gm1_W, gm1_b, gm2_W, gm2_b, fcc_W, fcc_b, out_W, out_b):
    raise NotImplementedError("write your pallas kernel here")


# speedup vs baseline: 1.0000x; 1.0000x over previous
import jax
import jax.numpy as jnp
from jax.experimental import pallas as pl
from jax.experimental.pallas import tpu as pltpu

B = 64


def _gcn_conv(x, edge_index, W, b):
    n = x.shape[0]
    loop = jnp.arange(n, dtype=edge_index.dtype)
    src = jnp.concatenate([edge_index[0], loop])
    dst = jnp.concatenate([edge_index[1], loop])
    h = x @ W
    deg = jax.ops.segment_sum(jnp.ones_like(dst, dtype=h.dtype), dst, num_segments=n)
    dinv = jnp.where(deg > 0, jax.lax.rsqrt(jnp.maximum(deg, 1e-12)), 0.0)
    norm = dinv[src] * dinv[dst]
    msg = h[src] * norm[:, None]
    return jax.ops.segment_sum(msg, dst, num_segments=n) + b


def _global_mean_pool(x, batch, num_graphs):
    s = jax.ops.segment_sum(x, batch, num_segments=num_graphs)
    cnt = jax.ops.segment_sum(jnp.ones((x.shape[0],), x.dtype), batch, num_segments=num_graphs)
    return s / jnp.maximum(cnt, 1.0)[:, None]


def _head_body(comb_ref, fcc_W_ref, fcc_b_ref, out_W_ref, out_b_ref, o_ref):
    h = jnp.maximum(
        jnp.dot(comb_ref[...], fcc_W_ref[...], preferred_element_type=jnp.float32)
        + fcc_b_ref[...], 0.0)
    o = jnp.maximum(
        jnp.dot(h, out_W_ref[...], preferred_element_type=jnp.float32)
        + out_b_ref[...], 0.0)
    o_ref[...] = o


def _head(combined, fcc_W, fcc_b, out_W, out_b):
    out_W_p = jnp.zeros((128, 128), jnp.float32).at[:, :2].set(out_W)
    out_b_p = jnp.zeros((128,), jnp.float32).at[:2].set(out_b)
    o = pl.pallas_call(
        _head_body,
        out_shape=jax.ShapeDtypeStruct((B, 128), jnp.float32),
    )(combined, fcc_W, fcc_b.reshape(1, 128), out_W_p, out_b_p.reshape(1, 128))
    return o[:, :2]


def kernel(metadata_prot, metadata_mol, x_prot, edge_index_prot, batch_vector_prot,
           x_mol, edge_index_mol, batch_vector_mol, fc1_W, fc1_b, fc2_W, fc2_b,
           fcm1_W, fcm1_b, fcm2_W, fcm2_b, gp1_W, gp1_b, gp2_W, gp2_b,
           gm1_W, gm1_b, gm2_W, gm2_b, fcc_W, fcc_b, out_W, out_b):
    mp = jax.nn.relu(metadata_prot @ fc1_W + fc1_b)
    mp = jax.nn.relu(mp @ fc2_W + fc2_b)
    mm = jax.nn.relu(metadata_mol @ fcm1_W + fcm1_b)
    mm = jax.nn.relu(mm @ fcm2_W + fcm2_b)
    xp = jax.nn.relu(_gcn_conv(x_prot, edge_index_prot, gp1_W, gp1_b))
    xp = jax.nn.relu(_gcn_conv(xp, edge_index_prot, gp2_W, gp2_b))
    xp = _global_mean_pool(xp, batch_vector_prot, B)
    xm = jax.nn.relu(_gcn_conv(x_mol, edge_index_mol, gm1_W, gm1_b))
    xm = jax.nn.relu(_gcn_conv(xm, edge_index_mol, gm2_W, gm2_b))
    xm = _global_mean_pool(xm, batch_vector_mol, B)
    combined = jnp.concatenate([mp, mm, xp, xm], axis=1)
    return _head(combined, fcc_W, fcc_b, out_W, out_b)


# speedup vs baseline: 8.5595x; 8.5591x over previous
import functools

import jax
import jax.numpy as jnp
from jax import lax
from jax.experimental import pallas as pl
from jax.experimental.pallas import tpu as pltpu
from jax.experimental.pallas import tpu_sc as plsc

B = 64
NUM_CORES = 2
NUM_SUBCORES = 16
NW = NUM_CORES * NUM_SUBCORES
CHUNK = 128
CPS = 8
SUPER = CHUNK * CPS
SW = 8
CH = 2000


def _cdiv(a, b):
    return (a + b - 1) // b


@functools.lru_cache(maxsize=None)
def _sc_degree(n_acc, e_pad):
    g_per_w = e_pad // (NW * SUPER)
    zch = n_acc // NUM_SUBCORES
    mesh = plsc.VectorSubcoreMesh(core_axis_name="c", subcore_axis_name="s")

    @functools.partial(
        pl.kernel,
        out_type=jax.ShapeDtypeStruct((NUM_CORES, n_acc, SW), jnp.float32),
        mesh=mesh,
        compiler_params=pltpu.CompilerParams(use_tc_tiling_on_sc=False),
        scratch_types=[
            pltpu.VMEM((CPS, CHUNK), jnp.int32),
            pltpu.VMEM((CHUNK, SW), jnp.float32),
            pltpu.VMEM_SHARED((n_acc, SW), jnp.float32),
            pltpu.SemaphoreType.DMA,
        ],
    )
    def kern(dst2, ones_hbm, zeros, out, dst_v, ones_v, acc, ssem):
        cid = lax.axis_index("c")
        sid = lax.axis_index("s")
        wid = sid * NUM_CORES + cid
        pltpu.sync_copy(ones_hbm, ones_v)
        pltpu.sync_copy(zeros.at[pl.ds(sid * zch, zch)],
                        acc.at[pl.ds(sid * zch, zch)])
        plsc.subcore_barrier()

        def body(g, c):
            row0 = (wid * g_per_w + g) * CPS
            pltpu.sync_copy(dst2.at[pl.ds(row0, CPS)], dst_v)
            descs = [
                pltpu.async_copy(ones_v, acc.at[dst_v.at[j]], ssem, add=True)
                for j in range(CPS)
            ]
            for d in descs:
                d.wait()
            return c

        lax.fori_loop(0, g_per_w, body, 0)
        plsc.subcore_barrier()
        pltpu.sync_copy(acc.at[pl.ds(sid * zch, zch)],
                        out.at[cid, pl.ds(sid * zch, zch)])

    return kern


@functools.lru_cache(maxsize=None)
def _sc_prop(n_tab, n_acc, e_pad, num_slices, ow):
    g_per_w = e_pad // (NW * SUPER)
    zch = n_acc // NUM_SUBCORES
    mesh = plsc.VectorSubcoreMesh(core_axis_name="c", subcore_axis_name="s")

    @functools.partial(
        pl.kernel,
        out_type=jax.ShapeDtypeStruct((NUM_CORES, n_acc, ow), jnp.float32),
        mesh=mesh,
        compiler_params=pltpu.CompilerParams(use_tc_tiling_on_sc=False),
        scratch_types=[
            pltpu.VMEM((CPS, CHUNK), jnp.int32),
            pltpu.VMEM((CPS, CHUNK), jnp.int32),
            pltpu.VMEM((CPS, CHUNK, SW), jnp.float32),
            pltpu.VMEM_SHARED((n_acc, SW), jnp.float32),
            pltpu.SemaphoreType.DMA,
            pltpu.SemaphoreType.DMA,
        ],
    )
    def kern(*refs):
        tables = refs[:num_slices]
        src2, dst2, zeros, out = refs[num_slices:num_slices + 4]
        src_v, dst_v, rows_v, acc, gsem, ssem = refs[num_slices + 4:]
        cid = lax.axis_index("c")
        sid = lax.axis_index("s")
        wid = sid * NUM_CORES + cid

        for s in range(num_slices):
            pltpu.sync_copy(zeros.at[pl.ds(sid * zch, zch)],
                            acc.at[pl.ds(sid * zch, zch)])
            plsc.subcore_barrier()

            def body(g, c, s=s):
                row0 = (wid * g_per_w + g) * CPS
                pltpu.sync_copy(src2.at[pl.ds(row0, CPS)], src_v)
                pltpu.sync_copy(dst2.at[pl.ds(row0, CPS)], dst_v)
                gd = [
                    pltpu.async_copy(tables[s].at[src_v.at[j]], rows_v.at[j], gsem)
                    for j in range(CPS)
                ]
                for d in gd:
                    d.wait()
                sd = [
                    pltpu.async_copy(rows_v.at[j], acc.at[dst_v.at[j]], ssem, add=True)
                    for j in range(CPS)
                ]
                for d in sd:
                    d.wait()
                return c

            lax.fori_loop(0, g_per_w, body, 0)
            plsc.subcore_barrier()
            pltpu.sync_copy(acc.at[pl.ds(sid * zch, zch)],
                            out.at[cid, pl.ds(sid * zch, zch), pl.ds(s * SW, SW)])
            plsc.subcore_barrier()

    return kern


def _meta_body(mdp, w1, b1, w2, b2, mdm, wm1, bm1, wm2, bm2, mp_ref, mm_ref):
    h = jnp.maximum(
        jnp.dot(mdp[...], w1[...], preferred_element_type=jnp.float32) + b1[...], 0.0)
    mp_ref[...] = jnp.maximum(
        jnp.dot(h, w2[...], preferred_element_type=jnp.float32) + b2[...], 0.0)
    hm = jnp.maximum(
        jnp.dot(mdm[...], wm1[...], preferred_element_type=jnp.float32) + bm1[...], 0.0)
    mm_ref[...] = jnp.maximum(
        jnp.dot(hm, wm2[...], preferred_element_type=jnp.float32) + bm2[...], 0.0)


def _prep_body(deg, x16, dinv_ref, x1_ref):
    d8 = lax.rsqrt(deg[0] + deg[1] + 1.0)
    d = jnp.concatenate([d8, d8], axis=1)
    dinv_ref[...] = d
    x1_ref[...] = x16[...] * d


def _layer1_body(p, x1, dinv, w1, b1, y1p_ref):
    z = dinv[...] * (p[0] + p[1] + x1[...])
    y = jnp.maximum(
        jnp.dot(z, w1[...], preferred_element_type=jnp.float32) + b1[...], 0.0)
    y1p_ref[...] = y * dinv[..., :1]


def _layer2_pool_body(q, y1p, dinv, w2, b2, batch, out_ref, acc_scr, cnt_scr):
    i = pl.program_id(0)

    @pl.when(i == 0)
    def _():
        acc_scr[...] = jnp.zeros_like(acc_scr)
        cnt_scr[...] = jnp.zeros_like(cnt_scr)

    qf = dinv[..., :1] * (q[0] + q[1] + y1p[...])
    y2 = jnp.maximum(
        jnp.dot(qf, w2[...], preferred_element_type=jnp.float32) + b2[...], 0.0)
    bvec = batch[0, 0, :]
    oh = (bvec[None, :] == lax.broadcasted_iota(jnp.int32, (B, bvec.shape[0]), 0)
          ).astype(jnp.float32)
    acc_scr[...] += jnp.dot(oh, y2, preferred_element_type=jnp.float32)
    cnt_scr[...] += jnp.sum(oh, axis=1)[:, None]

    @pl.when(i == pl.num_programs(0) - 1)
    def _():
        out_ref[...] = acc_scr[...] / jnp.maximum(cnt_scr[...], 1.0)


def _head_body(mp, mm, xp, xm, fcc_W, fcc_b, out_W, out_b, o_ref):
    comb = jnp.concatenate([mp[...], mm[...], xp[...], xm[...]], axis=1)
    h = jnp.maximum(
        jnp.dot(comb, fcc_W[...], preferred_element_type=jnp.float32) + fcc_b[...], 0.0)
    o_ref[...] = jnp.maximum(
        jnp.dot(h, out_W[...], preferred_element_type=jnp.float32) + out_b[...], 0.0)


def _edge_setup(edge_index, n_nodes):
    e = edge_index.shape[1]
    e_pad = _cdiv(e, NW * SUPER) * (NW * SUPER)
    src = jnp.concatenate(
        [edge_index[0], jnp.zeros((e_pad - e,), jnp.int32)]).reshape(e_pad // CHUNK, CHUNK)
    dst = jnp.concatenate(
        [edge_index[1], jnp.full((e_pad - e,), n_nodes, jnp.int32)]).reshape(e_pad // CHUNK, CHUNK)
    return src, dst, e_pad


def _gcn_tower(x, edge_index, batch_vector, w1, b1, w2, b2):
    n = x.shape[0]
    n_acc = _cdiv(n + 1, SUPER) * SUPER
    src2, dst2, e_pad = _edge_setup(edge_index, n)
    zeros = jnp.zeros((n_acc, SW), jnp.float32)
    ones = jnp.ones((CHUNK, SW), jnp.float32)
    ngrid = n // CH

    deg_p = _sc_degree(n_acc, e_pad)(dst2, ones, zeros)

    x16 = jnp.pad(x, ((0, 0), (0, 16 - x.shape[1])))
    blk = lambda: pl.BlockSpec((CH, 16), lambda i: (i, 0))
    dinv, x1 = pl.pallas_call(
        _prep_body,
        grid=(ngrid,),
        in_specs=[pl.BlockSpec((NUM_CORES, CH, SW), lambda i: (0, i, 0)), blk()],
        out_specs=[blk(), blk()],
        out_shape=[jax.ShapeDtypeStruct((n, 16), jnp.float32)] * 2,
    )(deg_p, x16)

    p = _sc_prop(n, n_acc, e_pad, 2, 16)(x1[:, :SW], x1[:, SW:], src2, dst2, zeros)

    w1p = jnp.zeros((16, 128), jnp.float32).at[:x.shape[1]].set(w1)
    y1p = pl.pallas_call(
        _layer1_body,
        grid=(ngrid,),
        in_specs=[
            pl.BlockSpec((NUM_CORES, CH, 16), lambda i: (0, i, 0)),
            blk(), blk(),
            pl.BlockSpec((16, 128), lambda i: (0, 0)),
            pl.BlockSpec((1, 128), lambda i: (0, 0)),
        ],
        out_specs=pl.BlockSpec((CH, 128), lambda i: (i, 0)),
        out_shape=jax.ShapeDtypeStruct((n, 128), jnp.float32),
    )(p, x1, dinv, w1p, b1.reshape(1, 128))

    tables = [y1p[:, s * SW:(s + 1) * SW] for s in range(16)]
    q = _sc_prop(n, n_acc, e_pad, 16, 128)(*tables, src2, dst2, zeros)

    batch3 = batch_vector.reshape(ngrid, 1, CH)
    pooled = pl.pallas_call(
        _layer2_pool_body,
        grid=(ngrid,),
        in_specs=[
            pl.BlockSpec((NUM_CORES, CH, 128), lambda i: (0, i, 0)),
            pl.BlockSpec((CH, 128), lambda i: (i, 0)),
            blk(),
            pl.BlockSpec((128, 128), lambda i: (0, 0)),
            pl.BlockSpec((1, 128), lambda i: (0, 0)),
            pl.BlockSpec((1, 1, CH), lambda i: (i, 0, 0)),
        ],
        out_specs=pl.BlockSpec((B, 128), lambda i: (0, 0)),
        out_shape=jax.ShapeDtypeStruct((B, 128), jnp.float32),
        scratch_shapes=[
            pltpu.VMEM((B, 128), jnp.float32),
            pltpu.VMEM((B, 128), jnp.float32),
        ],
    )(q, y1p, dinv, w2, b2.reshape(1, 128), batch3)
    return pooled


def kernel(metadata_prot, metadata_mol, x_prot, edge_index_prot, batch_vector_prot,
           x_mol, edge_index_mol, batch_vector_mol, fc1_W, fc1_b, fc2_W, fc2_b,
           fcm1_W, fcm1_b, fcm2_W, fcm2_b, gp1_W, gp1_b, gp2_W, gp2_b,
           gm1_W, gm1_b, gm2_W, gm2_b, fcc_W, fcc_b, out_W, out_b):
    mdm = jnp.pad(metadata_mol, ((0, 0), (0, 32 - 21)))
    wm1 = jnp.pad(fcm1_W, ((0, 32 - 21), (0, 0)))
    mp, mm = pl.pallas_call(
        _meta_body,
        out_shape=[jax.ShapeDtypeStruct((B, 128), jnp.float32),
                   jax.ShapeDtypeStruct((B, 64), jnp.float32)],
    )(metadata_prot, fc1_W, fc1_b.reshape(1, 128), fc2_W, fc2_b.reshape(1, 128),
      mdm, wm1, fcm1_b.reshape(1, 64), fcm2_W, fcm2_b.reshape(1, 64))

    xp = _gcn_tower(x_prot, edge_index_prot, batch_vector_prot,
                    gp1_W, gp1_b, gp2_W, gp2_b)
    xm = _gcn_tower(x_mol, edge_index_mol, batch_vector_mol,
                    gm1_W, gm1_b, gm2_W, gm2_b)

    out_Wp = jnp.zeros((128, 128), jnp.float32).at[:, :2].set(out_W)
    out_bp = jnp.zeros((128,), jnp.float32).at[:2].set(out_b)
    o = pl.pallas_call(
        _head_body,
        out_shape=jax.ShapeDtypeStruct((B, 128), jnp.float32),
    )(mp, mm, xp, xm, fcc_W, fcc_b.reshape(1, 128), out_Wp, out_bp.reshape(1, 128))
    return o[:, :2]


# speedup vs baseline: 8.7000x; 1.0164x over previous
import functools

import jax
import jax.numpy as jnp
from jax import lax
from jax.experimental import pallas as pl
from jax.experimental.pallas import tpu as pltpu
from jax.experimental.pallas import tpu_sc as plsc

B = 64
NUM_CORES = 2
NUM_SUBCORES = 16
NW = NUM_CORES * NUM_SUBCORES
CHUNK = 128
CPS = 8
SUPER = CHUNK * CPS
SW = 8
CH = 2000


def _cdiv(a, b):
    return (a + b - 1) // b


@functools.lru_cache(maxsize=None)
def _sc_degree(n_acc, e_pad):
    g_per_w = e_pad // (NW * SUPER)
    zch = n_acc // NUM_SUBCORES
    mesh = plsc.VectorSubcoreMesh(core_axis_name="c", subcore_axis_name="s")

    @functools.partial(
        pl.kernel,
        out_type=jax.ShapeDtypeStruct((NUM_CORES, n_acc, SW), jnp.float32),
        mesh=mesh,
        compiler_params=pltpu.CompilerParams(use_tc_tiling_on_sc=False),
        scratch_types=[
            pltpu.VMEM((CPS, CHUNK), jnp.int32),
            pltpu.VMEM((CHUNK, SW), jnp.float32),
            pltpu.VMEM_SHARED((n_acc, SW), jnp.float32),
            pltpu.SemaphoreType.DMA,
        ],
    )
    def kern(dst2, ones_hbm, zeros, out, dst_v, ones_v, acc, ssem):
        cid = lax.axis_index("c")
        sid = lax.axis_index("s")
        wid = sid * NUM_CORES + cid
        pltpu.sync_copy(ones_hbm, ones_v)
        pltpu.sync_copy(zeros.at[pl.ds(sid * zch, zch)],
                        acc.at[pl.ds(sid * zch, zch)])
        plsc.subcore_barrier()

        def body(g, c):
            row0 = (wid * g_per_w + g) * CPS
            pltpu.sync_copy(dst2.at[pl.ds(row0, CPS)], dst_v)
            descs = [
                pltpu.async_copy(ones_v, acc.at[dst_v.at[j]], ssem, add=True)
                for j in range(CPS)
            ]
            for d in descs:
                d.wait()
            return c

        lax.fori_loop(0, g_per_w, body, 0)
        plsc.subcore_barrier()
        pltpu.sync_copy(acc.at[pl.ds(sid * zch, zch)],
                        out.at[cid, pl.ds(sid * zch, zch)])

    return kern


@functools.lru_cache(maxsize=None)
def _sc_prop(n_tab, n_acc, e_pad, num_slices, sw):
    g_per_w = e_pad // (NW * SUPER)
    zch = n_acc // NUM_SUBCORES
    mesh = plsc.VectorSubcoreMesh(core_axis_name="c", subcore_axis_name="s")

    @functools.partial(
        pl.kernel,
        out_type=jax.ShapeDtypeStruct((num_slices, NUM_CORES, n_acc, sw),
                                      jnp.float32),
        mesh=mesh,
        compiler_params=pltpu.CompilerParams(use_tc_tiling_on_sc=False),
        scratch_types=[
            pltpu.VMEM((CPS, CHUNK), jnp.int32),
            pltpu.VMEM((CPS, CHUNK), jnp.int32),
            pltpu.VMEM((CPS, CHUNK, sw), jnp.float32),
            pltpu.VMEM_SHARED((n_acc, sw), jnp.float32),
            pltpu.SemaphoreType.DMA,
            pltpu.SemaphoreType.DMA,
        ],
    )
    def kern(*refs):
        tables = refs[:num_slices]
        src2, dst2, zeros, out = refs[num_slices:num_slices + 4]
        src_v, dst_v, rows_v, acc, gsem, ssem = refs[num_slices + 4:]
        cid = lax.axis_index("c")
        sid = lax.axis_index("s")
        wid = sid * NUM_CORES + cid

        for s in range(num_slices):
            pltpu.sync_copy(zeros.at[pl.ds(sid * zch, zch)],
                            acc.at[pl.ds(sid * zch, zch)])
            plsc.subcore_barrier()

            def body(g, c, s=s):
                row0 = (wid * g_per_w + g) * CPS
                pltpu.sync_copy(src2.at[pl.ds(row0, CPS)], src_v)
                pltpu.sync_copy(dst2.at[pl.ds(row0, CPS)], dst_v)
                gd = [
                    pltpu.async_copy(tables[s].at[src_v.at[j]], rows_v.at[j], gsem)
                    for j in range(CPS)
                ]
                for d in gd:
                    d.wait()
                sd = [
                    pltpu.async_copy(rows_v.at[j], acc.at[dst_v.at[j]], ssem, add=True)
                    for j in range(CPS)
                ]
                for d in sd:
                    d.wait()
                return c

            lax.fori_loop(0, g_per_w, body, 0)
            plsc.subcore_barrier()
            pltpu.sync_copy(acc.at[pl.ds(sid * zch, zch)],
                            out.at[s, cid, pl.ds(sid * zch, zch)])
            plsc.subcore_barrier()

    return kern


def _meta_body(mdp, w1, b1, w2, b2, mdm, wm1, bm1, wm2, bm2, mp_ref, mm_ref):
    h = jnp.maximum(
        jnp.dot(mdp[...], w1[...], preferred_element_type=jnp.float32) + b1[...], 0.0)
    mp_ref[...] = jnp.maximum(
        jnp.dot(h, w2[...], preferred_element_type=jnp.float32) + b2[...], 0.0)
    hm = jnp.maximum(
        jnp.dot(mdm[...], wm1[...], preferred_element_type=jnp.float32) + bm1[...], 0.0)
    mm_ref[...] = jnp.maximum(
        jnp.dot(hm, wm2[...], preferred_element_type=jnp.float32) + bm2[...], 0.0)


def _prep_body(deg, x16, dinv_ref, x1_ref):
    d8 = lax.rsqrt(deg[0] + deg[1] + 1.0)
    d = jnp.concatenate([d8, d8], axis=1)
    dinv_ref[...] = d
    x1_ref[...] = x16[...] * d


def _layer1_body(p, x1, dinv, w1, b1, y1p_ref):
    z = dinv[...] * (p[0] + p[1] + x1[...])
    y = jnp.maximum(
        jnp.dot(z, w1[...], preferred_element_type=jnp.float32) + b1[...], 0.0)
    y1p_ref[...] = y * dinv[..., :1]


def _layer2_pool_body(q, y1p, dinv, w2, b2, batch, out_ref, acc_scr, cnt_scr):
    i = pl.program_id(0)

    @pl.when(i == 0)
    def _():
        acc_scr[...] = jnp.zeros_like(acc_scr)
        cnt_scr[...] = jnp.zeros_like(cnt_scr)

    qf = dinv[..., :1] * (q[0] + q[1] + y1p[...])
    y2 = jnp.maximum(
        jnp.dot(qf, w2[...], preferred_element_type=jnp.float32) + b2[...], 0.0)
    bvec = batch[0, 0, :]
    oh = (bvec[None, :] == lax.broadcasted_iota(jnp.int32, (B, bvec.shape[0]), 0)
          ).astype(jnp.float32)
    acc_scr[...] += jnp.dot(oh, y2, preferred_element_type=jnp.float32)
    cnt_scr[...] += jnp.sum(oh, axis=1)[:, None]

    @pl.when(i == pl.num_programs(0) - 1)
    def _():
        out_ref[...] = acc_scr[...] / jnp.maximum(cnt_scr[...], 1.0)


def _head_body(mp, mm, xp, xm, fcc_W, fcc_b, out_W, out_b, o_ref):
    comb = jnp.concatenate([mp[...], mm[...], xp[...], xm[...]], axis=1)
    h = jnp.maximum(
        jnp.dot(comb, fcc_W[...], preferred_element_type=jnp.float32) + fcc_b[...], 0.0)
    o_ref[...] = jnp.maximum(
        jnp.dot(h, out_W[...], preferred_element_type=jnp.float32) + out_b[...], 0.0)


def _edge_setup(edge_index, n_nodes):
    e = edge_index.shape[1]
    e_pad = _cdiv(e, NW * SUPER) * (NW * SUPER)
    src = jnp.concatenate(
        [edge_index[0], jnp.zeros((e_pad - e,), jnp.int32)]).reshape(e_pad // CHUNK, CHUNK)
    dst = jnp.concatenate(
        [edge_index[1], jnp.full((e_pad - e,), n_nodes, jnp.int32)]).reshape(e_pad // CHUNK, CHUNK)
    return src, dst, e_pad


def _gcn_tower(x, edge_index, batch_vector, w1, b1, w2, b2, sw):
    n = x.shape[0]
    n_acc = _cdiv(n + 1, SUPER) * SUPER
    src2, dst2, e_pad = _edge_setup(edge_index, n)
    zeros = jnp.zeros((n_acc, SW), jnp.float32)
    zeros_s = zeros if sw == SW else jnp.zeros((n_acc, sw), jnp.float32)
    ones = jnp.ones((CHUNK, SW), jnp.float32)
    ngrid = n // CH

    deg_p = _sc_degree(n_acc, e_pad)(dst2, ones, zeros)

    x16 = jnp.pad(x, ((0, 0), (0, 16 - x.shape[1])))
    blk = lambda: pl.BlockSpec((CH, 16), lambda i: (i, 0))
    dinv, x1 = pl.pallas_call(
        _prep_body,
        grid=(ngrid,),
        in_specs=[pl.BlockSpec((NUM_CORES, CH, SW), lambda i: (0, i, 0)), blk()],
        out_specs=[blk(), blk()],
        out_shape=[jax.ShapeDtypeStruct((n, 16), jnp.float32)] * 2,
    )(deg_p, x16)

    nsl1 = 16 // sw
    t1 = [x1[:, i * sw:(i + 1) * sw] for i in range(nsl1)]
    p_s = _sc_prop(n, n_acc, e_pad, nsl1, sw)(*t1, src2, dst2, zeros_s)
    p = jnp.transpose(p_s, (1, 2, 0, 3)).reshape(NUM_CORES, n_acc, 16)

    w1p = jnp.zeros((16, 128), jnp.float32).at[:x.shape[1]].set(w1)
    y1p = pl.pallas_call(
        _layer1_body,
        grid=(ngrid,),
        in_specs=[
            pl.BlockSpec((NUM_CORES, CH, 16), lambda i: (0, i, 0)),
            blk(), blk(),
            pl.BlockSpec((16, 128), lambda i: (0, 0)),
            pl.BlockSpec((1, 128), lambda i: (0, 0)),
        ],
        out_specs=pl.BlockSpec((CH, 128), lambda i: (i, 0)),
        out_shape=jax.ShapeDtypeStruct((n, 128), jnp.float32),
    )(p, x1, dinv, w1p, b1.reshape(1, 128))

    nsl2 = 128 // sw
    tables = [y1p[:, s * sw:(s + 1) * sw] for s in range(nsl2)]
    q_s = _sc_prop(n, n_acc, e_pad, nsl2, sw)(*tables, src2, dst2, zeros_s)
    q = jnp.transpose(q_s, (1, 2, 0, 3)).reshape(NUM_CORES, n_acc, 128)

    batch3 = batch_vector.reshape(ngrid, 1, CH)
    pooled = pl.pallas_call(
        _layer2_pool_body,
        grid=(ngrid,),
        in_specs=[
            pl.BlockSpec((NUM_CORES, CH, 128), lambda i: (0, i, 0)),
            pl.BlockSpec((CH, 128), lambda i: (i, 0)),
            blk(),
            pl.BlockSpec((128, 128), lambda i: (0, 0)),
            pl.BlockSpec((1, 128), lambda i: (0, 0)),
            pl.BlockSpec((1, 1, CH), lambda i: (i, 0, 0)),
        ],
        out_specs=pl.BlockSpec((B, 128), lambda i: (0, 0)),
        out_shape=jax.ShapeDtypeStruct((B, 128), jnp.float32),
        scratch_shapes=[
            pltpu.VMEM((B, 128), jnp.float32),
            pltpu.VMEM((B, 128), jnp.float32),
        ],
    )(q, y1p, dinv, w2, b2.reshape(1, 128), batch3)
    return pooled


def kernel(metadata_prot, metadata_mol, x_prot, edge_index_prot, batch_vector_prot,
           x_mol, edge_index_mol, batch_vector_mol, fc1_W, fc1_b, fc2_W, fc2_b,
           fcm1_W, fcm1_b, fcm2_W, fcm2_b, gp1_W, gp1_b, gp2_W, gp2_b,
           gm1_W, gm1_b, gm2_W, gm2_b, fcc_W, fcc_b, out_W, out_b):
    mdm = jnp.pad(metadata_mol, ((0, 0), (0, 32 - 21)))
    wm1 = jnp.pad(fcm1_W, ((0, 32 - 21), (0, 0)))
    mp, mm = pl.pallas_call(
        _meta_body,
        out_shape=[jax.ShapeDtypeStruct((B, 128), jnp.float32),
                   jax.ShapeDtypeStruct((B, 64), jnp.float32)],
    )(metadata_prot, fc1_W, fc1_b.reshape(1, 128), fc2_W, fc2_b.reshape(1, 128),
      mdm, wm1, fcm1_b.reshape(1, 64), fcm2_W, fcm2_b.reshape(1, 64))

    xp = _gcn_tower(x_prot, edge_index_prot, batch_vector_prot,
                    gp1_W, gp1_b, gp2_W, gp2_b, 8)
    xm = _gcn_tower(x_mol, edge_index_mol, batch_vector_mol,
                    gm1_W, gm1_b, gm2_W, gm2_b, 16)

    out_Wp = jnp.zeros((128, 128), jnp.float32).at[:, :2].set(out_W)
    out_bp = jnp.zeros((128,), jnp.float32).at[:2].set(out_b)
    o = pl.pallas_call(
        _head_body,
        out_shape=jax.ShapeDtypeStruct((B, 128), jnp.float32),
    )(mp, mm, xp, xm, fcc_W, fcc_b.reshape(1, 128), out_Wp, out_bp.reshape(1, 128))
    return o[:, :2]
